# baseline (speedup 1.0000x reference)
import functools

import jax
import jax.numpy as jnp
from jax import lax
from jax.experimental import pallas as pl
from jax.experimental.pallas import tpu as pltpu
from jax.experimental.pallas import tpu_sc as plsc

B = 16384
D = 32
V = 1000000
NC = 2
NS = 16
NW = NC * NS
BPW = B // NW
CH = 128
NCH = BPW // CH
LANES = 16

NT = 7813
ROW_STRIDE = NT * 128
TILES_PER_TABLE = 4 * NT
TOTAL_TILES = 2 * TILES_PER_TABLE
TILES_PER_WORKER = -(-TOTAL_TILES // NW)
DEPTH = 8

_mesh = plsc.VectorSubcoreMesh(core_axis_name="c", subcore_axis_name="s")


@functools.partial(
    pl.kernel,
    out_type=(
        jax.ShapeDtypeStruct((D, NT, 128), jnp.float32),
        jax.ShapeDtypeStruct((D, NT, 128), jnp.float32),
    ),
    mesh=_mesh,
    scratch_types=[pltpu.SemaphoreType.DMA],
)
def _detile(tabu_hbm, tabi_hbm, linu_hbm, lini_hbm, sem):
    c = lax.axis_index("c")
    s = lax.axis_index("s")
    wid = s * NC + c

    def drain_one():
        pltpu.make_async_copy(
            tabu_hbm.at[pl.ds(0, 8), pl.ds(0, 128)],
            linu_hbm.at[pl.ds(0, 8), 0, :],
            sem).wait()

    def body(g, carry):
        t = (g * NW + wid) % TOTAL_TILES
        t2 = t % TILES_PER_TABLE
        tr = t2 // NT
        tc = t2 % NT

        @pl.when(t < TILES_PER_TABLE)
        def _():
            pltpu.async_copy(
                tabu_hbm.at[pl.ds(tr * 8, 8),
                            pl.ds(pl.multiple_of(tc * 128, 128), 128)],
                linu_hbm.at[pl.ds(tr * 8, 8), tc, :],
                sem)

        @pl.when(t >= TILES_PER_TABLE)
        def _():
            pltpu.async_copy(
                tabi_hbm.at[pl.ds(tr * 8, 8),
                            pl.ds(pl.multiple_of(tc * 128, 128), 128)],
                lini_hbm.at[pl.ds(tr * 8, 8), tc, :],
                sem)

        @pl.when(g >= DEPTH)
        def _():
            drain_one()

        return carry

    lax.fori_loop(0, TILES_PER_WORKER, body, 0)
    for _ in range(DEPTH):
        drain_one()


@functools.partial(
    pl.kernel,
    out_type=(
        jax.ShapeDtypeStruct((NW, LANES), jnp.float32),
        jax.ShapeDtypeStruct((B,), jnp.float32),
    ),
    mesh=_mesh,
    scratch_types=[
        pltpu.VMEM((B // CH, CH), jnp.int32),
        pltpu.VMEM((B // CH, CH), jnp.int32),
        pltpu.VMEM((NCH, CH), jnp.int32),
        pltpu.VMEM((NCH, CH), jnp.int32),
        pltpu.VMEM((B,), jnp.float32),
        pltpu.VMEM((B,), jnp.float32),
        pltpu.VMEM((BPW,), jnp.float32),
        pltpu.VMEM((BPW,), jnp.float32),
        pltpu.VMEM((BPW,), jnp.float32),
        pltpu.VMEM((LANES,), jnp.float32),
        pltpu.SemaphoreType.DMA,
    ],
    compiler_params=pltpu.CompilerParams(use_tc_tiling_on_sc=False),
)
def _gather_dot(gu_hbm, gi_hbm, uidx_hbm, iidx_hbm, ulin_hbm, ilin_hbm,
                ubias_hbm, ibias_hbm,
                part_out, bsum_out,
                gu_v, gi_v, uidx_v, iidx_v, ubuf, ibuf, ub_v, ib_v, bs_v,
                acc_v, sem):
    c = lax.axis_index("c")
    s = lax.axis_index("s")
    wid = s * NC + c
    base = wid * BPW
    nrows = B // CH

    pltpu.sync_copy(gu_hbm.at[pl.ds(wid * nrows, nrows), :], gu_v)
    pltpu.sync_copy(gi_hbm.at[pl.ds(wid * nrows, nrows), :], gi_v)
    pltpu.sync_copy(uidx_hbm.at[pl.ds(wid * NCH, NCH), :], uidx_v)
    pltpu.sync_copy(iidx_hbm.at[pl.ds(wid * NCH, NCH), :], iidx_v)

    copies = []
    for r in range(nrows):
        copies.append(pltpu.async_copy(
            ulin_hbm.at[gu_v.at[r]], ubuf.at[pl.ds(r * CH, CH)], sem))
        copies.append(pltpu.async_copy(
            ilin_hbm.at[gi_v.at[r]], ibuf.at[pl.ds(r * CH, CH)], sem))
    for j in range(NCH):
        copies.append(pltpu.async_copy(
            ubias_hbm.at[uidx_v.at[j]], ub_v.at[pl.ds(j * CH, CH)], sem))
        copies.append(pltpu.async_copy(
            ibias_hbm.at[iidx_v.at[j]], ib_v.at[pl.ds(j * CH, CH)], sem))
    for cp in copies:
        cp.wait()

    def bias_body(j, carry):
        sl = pl.ds(j * LANES, LANES)
        bs_v[sl] = ub_v[sl] + ib_v[sl]
        return carry

    lax.fori_loop(0, BPW // LANES, bias_body, 0)
    pltpu.sync_copy(bs_v, bsum_out.at[pl.ds(base, BPW)])

    def dot_body(i, acc):
        sl = pl.ds(i * LANES, LANES)
        return acc + ubuf[sl] * ibuf[sl]

    acc = lax.fori_loop(0, B // LANES, dot_body,
                        jnp.zeros((LANES,), jnp.float32))
    acc_v[pl.ds(0, LANES)] = acc
    pltpu.sync_copy(acc_v, part_out.at[wid])


def _tc_finish_body(part_ref, bs_ref, o_ref):
    total = jnp.sum(part_ref[...])
    o_ref[...] = 1.0 / (1.0 + jnp.exp(-(bs_ref[...] + total)))


_tc_finish = pl.pallas_call(
    _tc_finish_body,
    out_shape=jax.ShapeDtypeStruct((128, 128), jnp.float32),
)


def kernel(inputs, user_embedding, user_bias, item_embedding, item_bias):
    idx = inputs.astype(jnp.int32)
    ui = idx[:, 0]
    ii = idx[:, 1]
    linu, lini = _detile(user_embedding.T, item_embedding.T)
    c_off = (jnp.arange(D, dtype=jnp.int32) * ROW_STRIDE)[None, :]
    gu = (ui[:, None] + c_off).reshape(B * D // CH, CH)
    gi = (ii[:, None] + c_off).reshape(B * D // CH, CH)
    part, bsum = _gather_dot(
        gu, gi, ui.reshape(B // CH, CH), ii.reshape(B // CH, CH),
        linu.reshape(-1), lini.reshape(-1),
        user_bias.reshape(-1), item_bias.reshape(-1))
    out = _tc_finish(part, bsum.reshape(128, 128))
    return out.reshape(B, 1)


# speedup vs baseline: 39.4985x; 39.4985x over previous
import functools

import jax
import jax.numpy as jnp
from jax import lax
from jax.experimental import pallas as pl
from jax.experimental.pallas import tpu as pltpu
from jax.experimental.pallas import tpu_sc as plsc

B = 16384
D = 32
V = 1000000
NC = 2
NS = 16
NW = NC * NS
BPW = B // NW
CH = 128
NCH = BPW // CH
LANES = 16

NT = 7813
NT2 = 7816
ROW_STRIDE = NT2 * 128
KT = 16
CPG = -(-NT // KT)
UNITS = 2 * 4 * CPG
UPW = -(-UNITS // NW)

_mesh = plsc.VectorSubcoreMesh(core_axis_name="c", subcore_axis_name="s")


@functools.partial(
    pl.kernel,
    out_type=(
        jax.ShapeDtypeStruct((D, NT2, 128), jnp.float32),
        jax.ShapeDtypeStruct((D, NT2, 128), jnp.float32),
    ),
    mesh=_mesh,
    scratch_types=[
        pltpu.VMEM((2, 8, KT, 128), jnp.float32),
        pltpu.SemaphoreType.DMA,
        pltpu.SemaphoreType.DMA,
        pltpu.SemaphoreType.DMA,
    ],
)
def _detile(tabu_hbm, tabi_hbm, linu_hbm, lini_hbm, ring, sem_r0, sem_r1,
            sem_w):
    c = lax.axis_index("c")
    s = lax.axis_index("s")
    wid = s * NC + c

    def decode(u):
        table = u // (4 * CPG)
        r = u % (4 * CPG)
        tr = r // CPG
        ch = r % CPG
        tc0 = pl.multiple_of(jnp.minimum(ch * KT, NT2 - KT), 8)
        return table, tr, tc0

    def drain(sem, n):
        for _ in range(n):
            pltpu.make_async_copy(
                tabu_hbm.at[pl.ds(0, 8), pl.ds(0, 128)],
                ring.at[0, :, 0, :],
                sem).wait()

    def fire_reads(u, buf, sem):
        table, tr, tc0 = decode(u)
        for k in range(KT):
            col = pl.multiple_of(
                jnp.minimum((tc0 + k) * 128, (NT - 1) * 128), 128)

            @pl.when(table == 0)
            def _():
                pltpu.async_copy(
                    tabu_hbm.at[pl.ds(tr * 8, 8), pl.ds(col, 128)],
                    ring.at[buf, :, k, :], sem)

            @pl.when(table == 1)
            def _():
                pltpu.async_copy(
                    tabi_hbm.at[pl.ds(tr * 8, 8), pl.ds(col, 128)],
                    ring.at[buf, :, k, :], sem)

    def fire_write(u, buf):
        table, tr, tc0 = decode(u)

        @pl.when(table == 0)
        def _():
            pltpu.async_copy(
                ring.at[buf],
                linu_hbm.at[pl.ds(tr * 8, 8), pl.ds(tc0, KT), :], sem_w)

        @pl.when(table == 1)
        def _():
            pltpu.async_copy(
                ring.at[buf],
                lini_hbm.at[pl.ds(tr * 8, 8), pl.ds(tc0, KT), :], sem_w)

    def unit(g):
        return (wid * UPW + g) % UNITS

    def body(g, carry):
        @pl.when(g >= 2)
        def _():
            drain(sem_w, KT)

        @pl.when(g < UPW)
        def _():
            @pl.when(g % 2 == 0)
            def _():
                fire_reads(unit(g), g % 2, sem_r0)

            @pl.when(g % 2 == 1)
            def _():
                fire_reads(unit(g), g % 2, sem_r1)

        @pl.when(g >= 1)
        def _():
            @pl.when(g % 2 == 1)
            def _():
                drain(sem_r0, KT)
            @pl.when(g % 2 == 0)
            def _():
                drain(sem_r1, KT)
            fire_write(unit(g - 1), (g - 1) % 2)

        return carry

    lax.fori_loop(0, UPW + 1, body, 0)
    drain(sem_w, KT)


@functools.partial(
    pl.kernel,
    out_type=(
        jax.ShapeDtypeStruct((NW, LANES), jnp.float32),
        jax.ShapeDtypeStruct((B,), jnp.float32),
    ),
    mesh=_mesh,
    scratch_types=[
        pltpu.VMEM((B // CH, CH), jnp.int32),
        pltpu.VMEM((B // CH, CH), jnp.int32),
        pltpu.VMEM((NCH, CH), jnp.int32),
        pltpu.VMEM((NCH, CH), jnp.int32),
        pltpu.VMEM((B,), jnp.float32),
        pltpu.VMEM((B,), jnp.float32),
        pltpu.VMEM((BPW,), jnp.float32),
        pltpu.VMEM((BPW,), jnp.float32),
        pltpu.VMEM((BPW,), jnp.float32),
        pltpu.VMEM((LANES,), jnp.float32),
        pltpu.SemaphoreType.DMA,
    ],
    compiler_params=pltpu.CompilerParams(use_tc_tiling_on_sc=False),
)
def _gather_dot(gu_hbm, gi_hbm, uidx_hbm, iidx_hbm, ulin_hbm, ilin_hbm,
                ubias_hbm, ibias_hbm,
                part_out, bsum_out,
                gu_v, gi_v, uidx_v, iidx_v, ubuf, ibuf, ub_v, ib_v, bs_v,
                acc_v, sem):
    c = lax.axis_index("c")
    s = lax.axis_index("s")
    wid = s * NC + c
    base = wid * BPW
    nrows = B // CH

    pltpu.sync_copy(gu_hbm.at[pl.ds(wid * nrows, nrows), :], gu_v)
    pltpu.sync_copy(gi_hbm.at[pl.ds(wid * nrows, nrows), :], gi_v)
    pltpu.sync_copy(uidx_hbm.at[pl.ds(wid * NCH, NCH), :], uidx_v)
    pltpu.sync_copy(iidx_hbm.at[pl.ds(wid * NCH, NCH), :], iidx_v)

    copies = []
    for r in range(nrows):
        copies.append(pltpu.async_copy(
            ulin_hbm.at[gu_v.at[r]], ubuf.at[pl.ds(r * CH, CH)], sem))
        copies.append(pltpu.async_copy(
            ilin_hbm.at[gi_v.at[r]], ibuf.at[pl.ds(r * CH, CH)], sem))
    for j in range(NCH):
        copies.append(pltpu.async_copy(
            ubias_hbm.at[uidx_v.at[j]], ub_v.at[pl.ds(j * CH, CH)], sem))
        copies.append(pltpu.async_copy(
            ibias_hbm.at[iidx_v.at[j]], ib_v.at[pl.ds(j * CH, CH)], sem))
    for cp in copies:
        cp.wait()

    def bias_body(j, carry):
        sl = pl.ds(j * LANES, LANES)
        bs_v[sl] = ub_v[sl] + ib_v[sl]
        return carry

    lax.fori_loop(0, BPW // LANES, bias_body, 0)
    pltpu.sync_copy(bs_v, bsum_out.at[pl.ds(base, BPW)])

    def dot_body(i, acc):
        sl = pl.ds(i * LANES, LANES)
        return acc + ubuf[sl] * ibuf[sl]

    acc = lax.fori_loop(0, B // LANES, dot_body,
                        jnp.zeros((LANES,), jnp.float32))
    acc_v[pl.ds(0, LANES)] = acc
    pltpu.sync_copy(acc_v, part_out.at[wid])


def _tc_finish_body(part_ref, bs_ref, o_ref):
    total = jnp.sum(part_ref[...])
    o_ref[...] = 1.0 / (1.0 + jnp.exp(-(bs_ref[...] + total)))


_tc_finish = pl.pallas_call(
    _tc_finish_body,
    out_shape=jax.ShapeDtypeStruct((128, 128), jnp.float32),
)


def kernel(inputs, user_embedding, user_bias, item_embedding, item_bias):
    idx = inputs.astype(jnp.int32)
    ui = idx[:, 0]
    ii = idx[:, 1]
    linu, lini = _detile(user_embedding.T, item_embedding.T)
    c_off = (jnp.arange(D, dtype=jnp.int32) * ROW_STRIDE)[None, :]
    gu = (ui[:, None] + c_off).reshape(B * D // CH, CH)
    gi = (ii[:, None] + c_off).reshape(B * D // CH, CH)
    part, bsum = _gather_dot(
        gu, gi, ui.reshape(B // CH, CH), ii.reshape(B // CH, CH),
        linu.reshape(-1), lini.reshape(-1),
        user_bias.reshape(-1), item_bias.reshape(-1))
    out = _tc_finish(part, bsum.reshape(128, 128))
    return out.reshape(B, 1)
